# baseline (speedup 1.0000x reference)
import jax
import jax.numpy as jnp
from jax import lax
from jax.experimental import pallas as pl
from jax.experimental.pallas import tpu as pltpu
from jax.experimental.pallas import tpu_sc as plsc

_N = 50000
_E = 1600000
_ND = 250000
_NC, _NS, _L = 2, 16, 16
_NW = _NC * _NS
_ZPAD = 250880
_CH = _ZPAD // _NS
_EPT = _E // _NW
_K = 10000
_NB = _EPT // _K
_NPAD = 51200
_CHN = _NPAD // _NS
_KT = _K + _L

_mesh = plsc.VectorSubcoreMesh(core_axis_name="c", subcore_axis_name="s")


def _seg_body(pre_hbm, post_hbm, w_hbm, z_hbm, out_hbm,
              idx_v, post_v, w_v, g_v, zstage, zblk, zsp, accsp, sem):
    cid = lax.axis_index("c")
    sid = lax.axis_index("s")
    wid = sid * _NC + cid

    pltpu.sync_copy(z_hbm.at[pl.ds(sid * _CH, _CH)], zstage)
    pltpu.sync_copy(zstage, zsp.at[pl.ds(sid * _CH, _CH)])

    def _zero(i, c):
        zblk[pl.ds(pl.multiple_of(i * _L, _L), _L)] = jnp.zeros((_L,), jnp.float32)
        return c
    lax.fori_loop(0, _CHN // _L, _zero, 0)
    pltpu.sync_copy(zblk, accsp.at[pl.ds(sid * _CHN, _CHN)])
    tail = pl.ds(_K, _L)
    idx_v[tail] = jnp.zeros((_L,), jnp.int32)
    post_v[tail] = jnp.zeros((_L,), jnp.int32)
    w_v[tail] = jnp.zeros((_L,), jnp.float32)
    plsc.subcore_barrier()

    for b in range(_NB):
        base = wid * _EPT + b * _K
        pltpu.sync_copy(pre_hbm.at[pl.ds(base, _K)], idx_v.at[pl.ds(0, _K)])
        pltpu.sync_copy(post_hbm.at[pl.ds(base, _K)], post_v.at[pl.ds(0, _K)])
        pltpu.sync_copy(w_hbm.at[pl.ds(base, _K)], w_v.at[pl.ds(0, _K)])

        pltpu.async_copy(zsp.at[idx_v], g_v, sem).wait()

        def _mul(i, c):
            s = pl.ds(pl.multiple_of(i * _L, _L), _L)
            g_v[s] = g_v[s] * w_v[s]
            return c
        lax.fori_loop(0, _KT // _L, _mul, 0)

        pltpu.sync_copy(g_v, accsp.at[post_v], add=True)

    plsc.subcore_barrier()
    pltpu.sync_copy(accsp.at[pl.ds(sid * _CHN, _CHN)], zblk)
    pltpu.sync_copy(zblk, out_hbm.at[pl.ds(cid * _NPAD + sid * _CHN, _CHN)])


_seg_sum = pl.kernel(
    _seg_body,
    out_type=jax.ShapeDtypeStruct((_NC * _NPAD,), jnp.float32),
    mesh=_mesh,
    compiler_params=pltpu.CompilerParams(use_tc_tiling_on_sc=False, needs_layout_passes=False),
    scratch_types=[
        pltpu.VMEM((_KT,), jnp.int32),
        pltpu.VMEM((_KT,), jnp.int32),
        pltpu.VMEM((_KT,), jnp.float32),
        pltpu.VMEM((_KT,), jnp.float32),
        pltpu.VMEM((_CH,), jnp.float32),
        pltpu.VMEM((_CHN,), jnp.float32),
        pltpu.VMEM_SHARED((_ZPAD,), jnp.float32),
        pltpu.VMEM_SHARED((_NPAD,), jnp.float32),
        pltpu.SemaphoreType.DMA,
    ],
)


def _glif_body(p0_ref, p1_ref, v_ref, ext_ref, decay_ref, cf_ref,
               vth_ref, vreset_ref, el_ref, out_ref):
    rec = p0_ref[...] + p1_ref[...]
    new_v = decay_ref[...] * v_ref[...] + cf_ref[...] * (rec + ext_ref[...])
    v_scaled = (new_v - vth_ref[...]) / (vth_ref[...] - el_ref[...] + 1e-8)
    z = (v_scaled > 0.0).astype(jnp.float32)
    v_out = new_v * (1.0 - z) + vreset_ref[...] * z
    out_ref[0:1, :] = z
    out_ref[1:2, :] = v_out


def kernel(z_buf, v, ext_current, rec_weights, decay, current_factor,
           v_th, v_reset, e_l, rec_indices):
    zflat = jnp.pad(z_buf.reshape(-1), (0, _ZPAD - _ND))
    ri_flat = rec_indices.reshape(-1)
    partial = _seg_sum(ri_flat[1::2], ri_flat[0::2], rec_weights, zflat)
    p0 = partial[:_N][None, :]
    p1 = partial[_NPAD:_NPAD + _N][None, :]
    out2 = pl.pallas_call(
        _glif_body,
        out_shape=jax.ShapeDtypeStruct((2, _N), jnp.float32),
    )(p0, p1, v, ext_current, decay[None, :], current_factor[None, :],
      v_th[None, :], v_reset[None, :], e_l[None, :])
    return out2.reshape(1, 2 * _N)


# speedup vs baseline: 1.2393x; 1.2393x over previous
import jax
import jax.numpy as jnp
from jax import lax
from jax.experimental import pallas as pl
from jax.experimental.pallas import tpu as pltpu
from jax.experimental.pallas import tpu_sc as plsc

_N = 50000
_E = 1600000
_ND = 250000
_NC, _NS, _L = 2, 16, 16
_NW = _NC * _NS
_ZPAD = 250880
_CH = _ZPAD // _NS
_EPT = _E // _NW
_K = 10000
_NB = _EPT // _K
_NPAD = 51200
_CHN = _NPAD // _NS
_KT = _K + _L

_mesh = plsc.VectorSubcoreMesh(core_axis_name="c", subcore_axis_name="s")


def _seg_body(ri_hbm, w_hbm, z_hbm, out_hbm,
              ri_v, idx_v, post_v, w_v, g_v, odd_v, even_v,
              zblk, zsp, accsp, ristage, sem, sem2):
    cid = lax.axis_index("c")
    sid = lax.axis_index("s")
    wid = sid * _NC + cid

    for r in range(2):
        zo = sid * _CH + r * (_CH // 2)
        pltpu.sync_copy(z_hbm.at[pl.ds(zo, _CH // 2)], g_v.at[pl.ds(0, _CH // 2)])
        pltpu.sync_copy(g_v.at[pl.ds(0, _CH // 2)], zsp.at[pl.ds(zo, _CH // 2)])

    def _zero(i, c):
        zblk[pl.ds(pl.multiple_of(i * _L, _L), _L)] = jnp.zeros((_L,), jnp.float32)
        return c
    lax.fori_loop(0, _CHN // _L, _zero, 0)
    pltpu.sync_copy(zblk, accsp.at[pl.ds(sid * _CHN, _CHN)])

    lane2 = lax.iota(jnp.int32, _L) * 2
    def _mkidx(i, vec):
        o = pl.ds(pl.multiple_of(i * _L, _L), _L)
        odd_v[o] = vec
        even_v[o] = vec - 1
        return vec + 2 * _L
    lax.fori_loop(0, _KT // _L, _mkidx, sid * (2 * _KT) + 1 + lane2)

    tail = pl.ds(_K, _L)
    w_v[tail] = jnp.zeros((_L,), jnp.float32)
    plsc.subcore_barrier()

    zeros_i = jnp.zeros((_L,), jnp.int32)
    for b in range(_NB):
        base = wid * _EPT + b * _K
        pltpu.sync_copy(ri_hbm.at[pl.ds(2 * base, 2 * _K)], ri_v)
        pltpu.sync_copy(w_hbm.at[pl.ds(base, _K)], w_v.at[pl.ds(0, _K)])
        pltpu.sync_copy(ri_v, ristage.at[pl.ds(sid * (2 * _KT), 2 * _K)])

        cp1 = pltpu.async_copy(ristage.at[odd_v], idx_v, sem)
        cp2 = pltpu.async_copy(ristage.at[even_v], post_v, sem2)
        cp1.wait()
        cp2.wait()
        idx_v[tail] = zeros_i
        post_v[tail] = zeros_i

        pltpu.async_copy(zsp.at[idx_v], g_v, sem).wait()

        def _mul(i, c):
            s = pl.ds(pl.multiple_of(i * _L, _L), _L)
            g_v[s] = g_v[s] * w_v[s]
            return c
        lax.fori_loop(0, _KT // _L, _mul, 0)

        pltpu.sync_copy(g_v, accsp.at[post_v], add=True)

    plsc.subcore_barrier()
    pltpu.sync_copy(accsp.at[pl.ds(sid * _CHN, _CHN)], zblk)
    pltpu.sync_copy(zblk, out_hbm.at[pl.ds(cid * _NPAD + sid * _CHN, _CHN)])


_seg_sum = pl.kernel(
    _seg_body,
    out_type=jax.ShapeDtypeStruct((_NC * _NPAD,), jnp.float32),
    mesh=_mesh,
    compiler_params=pltpu.CompilerParams(use_tc_tiling_on_sc=False, needs_layout_passes=False),
    scratch_types=[
        pltpu.VMEM((2 * _K,), jnp.int32),
        pltpu.VMEM((_KT,), jnp.int32),
        pltpu.VMEM((_KT,), jnp.int32),
        pltpu.VMEM((_KT,), jnp.float32),
        pltpu.VMEM((_KT,), jnp.float32),
        pltpu.VMEM((_KT,), jnp.int32),
        pltpu.VMEM((_KT,), jnp.int32),
        pltpu.VMEM((_CHN,), jnp.float32),
        pltpu.VMEM_SHARED((_ZPAD,), jnp.float32),
        pltpu.VMEM_SHARED((_NPAD,), jnp.float32),
        pltpu.VMEM_SHARED((_NS * 2 * _KT,), jnp.int32),
        pltpu.SemaphoreType.DMA,
        pltpu.SemaphoreType.DMA,
    ],
)


def _glif_body(p0_ref, p1_ref, v_ref, ext_ref, decay_ref, cf_ref,
               vth_ref, vreset_ref, el_ref, out_ref):
    rec = p0_ref[...] + p1_ref[...]
    new_v = decay_ref[...] * v_ref[...] + cf_ref[...] * (rec + ext_ref[...])
    v_scaled = (new_v - vth_ref[...]) / (vth_ref[...] - el_ref[...] + 1e-8)
    z = (v_scaled > 0.0).astype(jnp.float32)
    v_out = new_v * (1.0 - z) + vreset_ref[...] * z
    out_ref[0:1, :] = z
    out_ref[1:2, :] = v_out


def kernel(z_buf, v, ext_current, rec_weights, decay, current_factor,
           v_th, v_reset, e_l, rec_indices):
    zflat = jnp.pad(z_buf.reshape(-1), (0, _ZPAD - _ND))
    partial = _seg_sum(rec_indices.reshape(-1), rec_weights, zflat)
    p0 = partial[:_N][None, :]
    p1 = partial[_NPAD:_NPAD + _N][None, :]
    out2 = pl.pallas_call(
        _glif_body,
        out_shape=jax.ShapeDtypeStruct((2, _N), jnp.float32),
    )(p0, p1, v, ext_current, decay[None, :], current_factor[None, :],
      v_th[None, :], v_reset[None, :], e_l[None, :])
    return out2.reshape(1, 2 * _N)


# speedup vs baseline: 16.0001x; 12.9110x over previous
import jax
import jax.numpy as jnp
from jax import lax
from jax.experimental import pallas as pl
from jax.experimental.pallas import tpu as pltpu
from jax.experimental.pallas import tpu_sc as plsc

_N = 50000
_E = 1600000
_ND = 250000
_NC, _NS, _L = 2, 16, 16
_NW = _NC * _NS
_ZPAD = 250880
_CH = _ZPAD // _NS
_EPT = _E // _NW
_K = 10000
_NB = _EPT // _K
_KT = _K + _L
_NPAD = 51200
_CHN = _NPAD // _NS

_mesh = plsc.VectorSubcoreMesh(core_axis_name="c", subcore_axis_name="s")


def _seg_body(pre_hbm, post_hbm, w_hbm, z_hbm, out_hbm,
              idx_v, post_v, w_v, g_v, zblk, zsp, accsp, sem):
    cid = lax.axis_index("c")
    sid = lax.axis_index("s")
    wid = sid * _NC + cid

    for r in range(2):
        zo = sid * _CH + r * (_CH // 2)
        pltpu.sync_copy(z_hbm.at[pl.ds(zo, _CH // 2)], g_v.at[pl.ds(0, _CH // 2)])
        pltpu.sync_copy(g_v.at[pl.ds(0, _CH // 2)], zsp.at[pl.ds(zo, _CH // 2)])

    def _zero(i, c):
        zblk[pl.ds(pl.multiple_of(i * _L, _L), _L)] = jnp.zeros((_L,), jnp.float32)
        return c
    lax.fori_loop(0, _CHN // _L, _zero, 0)
    pltpu.sync_copy(zblk, accsp.at[pl.ds(sid * _CHN, _CHN)])

    tail = pl.ds(_K, _L)
    zeros_i = jnp.zeros((_L,), jnp.int32)
    idx_v[tail] = zeros_i
    post_v[tail] = zeros_i
    w_v[tail] = jnp.zeros((_L,), jnp.float32)
    plsc.subcore_barrier()

    for b in range(_NB):
        base = wid * _EPT + b * _K
        pltpu.sync_copy(pre_hbm.at[pl.ds(base, _K)], idx_v.at[pl.ds(0, _K)])
        pltpu.sync_copy(post_hbm.at[pl.ds(base, _K)], post_v.at[pl.ds(0, _K)])
        pltpu.sync_copy(w_hbm.at[pl.ds(base, _K)], w_v.at[pl.ds(0, _K)])

        pltpu.async_copy(zsp.at[idx_v], g_v, sem).wait()

        def _mul(i, c):
            s = pl.ds(pl.multiple_of(i * _L, _L), _L)
            g_v[s] = g_v[s] * w_v[s]
            return c
        lax.fori_loop(0, _KT // _L, _mul, 0)

        pltpu.sync_copy(g_v, accsp.at[post_v], add=True)

    plsc.subcore_barrier()
    pltpu.sync_copy(accsp.at[pl.ds(sid * _CHN, _CHN)], zblk)
    pltpu.sync_copy(zblk, out_hbm.at[pl.ds(cid * _NPAD + sid * _CHN, _CHN)])


_seg_sum = pl.kernel(
    _seg_body,
    out_type=jax.ShapeDtypeStruct((_NC * _NPAD,), jnp.float32),
    mesh=_mesh,
    compiler_params=pltpu.CompilerParams(use_tc_tiling_on_sc=False,
                                         needs_layout_passes=False),
    scratch_types=[
        pltpu.VMEM((_KT,), jnp.int32),
        pltpu.VMEM((_KT,), jnp.int32),
        pltpu.VMEM((_KT,), jnp.float32),
        pltpu.VMEM((_KT,), jnp.float32),
        pltpu.VMEM((_CHN,), jnp.float32),
        pltpu.VMEM_SHARED((_ZPAD,), jnp.float32),
        pltpu.VMEM_SHARED((_NPAD,), jnp.float32),
        pltpu.SemaphoreType.DMA,
    ],
)


def _glif_body(p0_ref, p1_ref, v_ref, ext_ref, decay_ref, cf_ref,
               vth_ref, vreset_ref, el_ref, out_ref):
    rec = p0_ref[...] + p1_ref[...]
    new_v = decay_ref[...] * v_ref[...] + cf_ref[...] * (rec + ext_ref[...])
    v_scaled = (new_v - vth_ref[...]) / (vth_ref[...] - el_ref[...] + 1e-8)
    z = (v_scaled > 0.0).astype(jnp.float32)
    v_out = new_v * (1.0 - z) + vreset_ref[...] * z
    out_ref[0:1, :] = z
    out_ref[1:2, :] = v_out


def kernel(z_buf, v, ext_current, rec_weights, decay, current_factor,
           v_th, v_reset, e_l, rec_indices):
    pre = rec_indices[:, 1]
    post = rec_indices[:, 0]
    zflat = jnp.pad(z_buf.reshape(-1), (0, _ZPAD - _ND))
    partial = _seg_sum(pre, post, rec_weights, zflat)
    p0 = partial[:_N][None, :]
    p1 = partial[_NPAD:_NPAD + _N][None, :]
    out2 = pl.pallas_call(
        _glif_body,
        out_shape=jax.ShapeDtypeStruct((2, _N), jnp.float32),
    )(p0, p1, v, ext_current, decay[None, :], current_factor[None, :],
      v_th[None, :], v_reset[None, :], e_l[None, :])
    return out2.reshape(1, 2 * _N)


# speedup vs baseline: 21.6468x; 1.3529x over previous
import jax
import jax.numpy as jnp
from jax import lax
from jax.experimental import pallas as pl
from jax.experimental.pallas import tpu as pltpu
from jax.experimental.pallas import tpu_sc as plsc

_N = 50000
_E = 1600000
_ND = 250000
_NC, _NS, _L = 2, 16, 16
_NW = _NC * _NS
_ZPAD = 250880
_CH = _ZPAD // _NS
_EA = 800256
_EB = _E - _EA
_NPAD = 51200
_CHN = _NPAD // _NS

_mesh = plsc.VectorSubcoreMesh(core_axis_name="c", subcore_axis_name="s")


def _make_seg_sum(ept, k):
    nb = ept // k
    kt = k + _L

    def _seg_body(pre_hbm, post_hbm, w_hbm, z_hbm, out_hbm,
                  idx_v, post_v, w_v, g_v, zblk, zsp, accsp, sem):
        cid = lax.axis_index("c")
        sid = lax.axis_index("s")
        wid = sid * _NC + cid

        for r in range(2):
            zo = sid * _CH + r * (_CH // 2)
            pltpu.sync_copy(z_hbm.at[pl.ds(zo, _CH // 2)],
                            g_v.at[pl.ds(0, _CH // 2)])
            pltpu.sync_copy(g_v.at[pl.ds(0, _CH // 2)],
                            zsp.at[pl.ds(zo, _CH // 2)])

        def _zero(i, c):
            zblk[pl.ds(pl.multiple_of(i * _L, _L), _L)] = jnp.zeros((_L,), jnp.float32)
            return c
        lax.fori_loop(0, _CHN // _L, _zero, 0)
        pltpu.sync_copy(zblk, accsp.at[pl.ds(sid * _CHN, _CHN)])

        tail = pl.ds(k, _L)
        zeros_i = jnp.zeros((_L,), jnp.int32)
        idx_v[tail] = zeros_i
        post_v[tail] = zeros_i
        w_v[tail] = jnp.zeros((_L,), jnp.float32)
        plsc.subcore_barrier()

        for b in range(nb):
            base = wid * ept + b * k
            pltpu.sync_copy(pre_hbm.at[pl.ds(base, k)], idx_v.at[pl.ds(0, k)])
            pltpu.sync_copy(post_hbm.at[pl.ds(base, k)], post_v.at[pl.ds(0, k)])
            pltpu.sync_copy(w_hbm.at[pl.ds(base, k)], w_v.at[pl.ds(0, k)])

            pltpu.async_copy(zsp.at[idx_v], g_v, sem).wait()

            def _mul(i, c):
                s = pl.ds(pl.multiple_of(i * _L, _L), _L)
                g_v[s] = g_v[s] * w_v[s]
                return c
            lax.fori_loop(0, kt // _L, _mul, 0)

            pltpu.sync_copy(g_v, accsp.at[post_v], add=True)

        plsc.subcore_barrier()
        pltpu.sync_copy(accsp.at[pl.ds(sid * _CHN, _CHN)], zblk)
        pltpu.sync_copy(zblk, out_hbm.at[pl.ds(cid * _NPAD + sid * _CHN, _CHN)])

    gbuf = max(kt, _CH // 2)
    return pl.kernel(
        _seg_body,
        out_type=jax.ShapeDtypeStruct((_NC * _NPAD,), jnp.float32),
        mesh=_mesh,
        compiler_params=pltpu.CompilerParams(use_tc_tiling_on_sc=False,
                                             needs_layout_passes=False),
        scratch_types=[
            pltpu.VMEM((kt,), jnp.int32),
            pltpu.VMEM((kt,), jnp.int32),
            pltpu.VMEM((kt,), jnp.float32),
            pltpu.VMEM((gbuf,), jnp.float32),
            pltpu.VMEM((_CHN,), jnp.float32),
            pltpu.VMEM_SHARED((_ZPAD,), jnp.float32),
            pltpu.VMEM_SHARED((_NPAD,), jnp.float32),
            pltpu.SemaphoreType.DMA,
        ],
    )


_seg_sum_a = _make_seg_sum(_EA // _NW, 8336)
_seg_sum_b = _make_seg_sum(_EB // _NW, 12496)


def _glif_body(p0_ref, p1_ref, p2_ref, p3_ref, v_ref, ext_ref, decay_ref,
               cf_ref, vth_ref, vreset_ref, el_ref, out_ref):
    rec = (p0_ref[...] + p1_ref[...]) + (p2_ref[...] + p3_ref[...])
    new_v = decay_ref[...] * v_ref[...] + cf_ref[...] * (rec + ext_ref[...])
    v_scaled = (new_v - vth_ref[...]) / (vth_ref[...] - el_ref[...] + 1e-8)
    z = (v_scaled > 0.0).astype(jnp.float32)
    v_out = new_v * (1.0 - z) + vreset_ref[...] * z
    out_ref[0:1, :] = z
    out_ref[1:2, :] = v_out


def kernel(z_buf, v, ext_current, rec_weights, decay, current_factor,
           v_th, v_reset, e_l, rec_indices):
    zflat = jnp.pad(z_buf.reshape(-1), (0, _ZPAD - _ND))
    pa = _seg_sum_a(rec_indices[:_EA, 1], rec_indices[:_EA, 0],
                    rec_weights[:_EA], zflat)
    pb = _seg_sum_b(rec_indices[_EA:, 1], rec_indices[_EA:, 0],
                    rec_weights[_EA:], zflat)
    p0 = pa[:_N][None, :]
    p1 = pa[_NPAD:_NPAD + _N][None, :]
    p2 = pb[:_N][None, :]
    p3 = pb[_NPAD:_NPAD + _N][None, :]
    out2 = pl.pallas_call(
        _glif_body,
        out_shape=jax.ShapeDtypeStruct((2, _N), jnp.float32),
    )(p0, p1, p2, p3, v, ext_current, decay[None, :], current_factor[None, :],
      v_th[None, :], v_reset[None, :], e_l[None, :])
    return out2.reshape(1, 2 * _N)


# speedup vs baseline: 21.8556x; 1.0096x over previous
import jax
import jax.numpy as jnp
from jax import lax
from jax.experimental import pallas as pl
from jax.experimental.pallas import tpu as pltpu
from jax.experimental.pallas import tpu_sc as plsc

_N = 50000
_E = 1600000
_ND = 250000
_NC, _NS, _L = 2, 16, 16
_NW = _NC * _NS
_ZPAD = 250880
_CH = _ZPAD // _NS
_EA = 533504
_EB = 533504
_EC = _E - _EA - _EB
_NPAD = 51200
_CHN = _NPAD // _NS

_mesh = plsc.VectorSubcoreMesh(core_axis_name="c", subcore_axis_name="s")


def _make_seg_sum(ept, k):
    nb = ept // k
    kt = k + _L

    def _seg_body(pre_hbm, post_hbm, w_hbm, z_hbm, out_hbm,
                  idx_v, post_v, w_v, g_v, zblk, zsp, accsp, sem):
        cid = lax.axis_index("c")
        sid = lax.axis_index("s")
        wid = sid * _NC + cid

        for r in range(2):
            zo = sid * _CH + r * (_CH // 2)
            pltpu.sync_copy(z_hbm.at[pl.ds(zo, _CH // 2)],
                            g_v.at[pl.ds(0, _CH // 2)])
            pltpu.sync_copy(g_v.at[pl.ds(0, _CH // 2)],
                            zsp.at[pl.ds(zo, _CH // 2)])

        def _zero(i, c):
            zblk[pl.ds(pl.multiple_of(i * _L, _L), _L)] = jnp.zeros((_L,), jnp.float32)
            return c
        lax.fori_loop(0, _CHN // _L, _zero, 0)
        pltpu.sync_copy(zblk, accsp.at[pl.ds(sid * _CHN, _CHN)])

        tail = pl.ds(k, _L)
        zeros_i = jnp.zeros((_L,), jnp.int32)
        idx_v[tail] = zeros_i
        post_v[tail] = zeros_i
        w_v[tail] = jnp.zeros((_L,), jnp.float32)
        plsc.subcore_barrier()

        for b in range(nb):
            base = wid * ept + b * k
            pltpu.sync_copy(pre_hbm.at[pl.ds(base, k)], idx_v.at[pl.ds(0, k)])
            pltpu.sync_copy(post_hbm.at[pl.ds(base, k)], post_v.at[pl.ds(0, k)])
            pltpu.sync_copy(w_hbm.at[pl.ds(base, k)], w_v.at[pl.ds(0, k)])

            pltpu.async_copy(zsp.at[idx_v], g_v, sem).wait()

            def _mul(i, c):
                s = pl.ds(pl.multiple_of(i * _L, _L), _L)
                g_v[s] = g_v[s] * w_v[s]
                return c
            lax.fori_loop(0, kt // _L, _mul, 0)

            pltpu.sync_copy(g_v, accsp.at[post_v], add=True)

        plsc.subcore_barrier()
        pltpu.sync_copy(accsp.at[pl.ds(sid * _CHN, _CHN)], zblk)
        pltpu.sync_copy(zblk, out_hbm.at[pl.ds(cid * _NPAD + sid * _CHN, _CHN)])

    gbuf = max(kt, _CH // 2)
    return pl.kernel(
        _seg_body,
        out_type=jax.ShapeDtypeStruct((_NC * _NPAD,), jnp.float32),
        mesh=_mesh,
        compiler_params=pltpu.CompilerParams(use_tc_tiling_on_sc=False,
                                             needs_layout_passes=False),
        scratch_types=[
            pltpu.VMEM((kt,), jnp.int32),
            pltpu.VMEM((kt,), jnp.int32),
            pltpu.VMEM((kt,), jnp.float32),
            pltpu.VMEM((gbuf,), jnp.float32),
            pltpu.VMEM((_CHN,), jnp.float32),
            pltpu.VMEM_SHARED((_ZPAD,), jnp.float32),
            pltpu.VMEM_SHARED((_NPAD,), jnp.float32),
            pltpu.SemaphoreType.DMA,
        ],
    )


_seg_sum_a = _make_seg_sum(_EA // _NW, _EA // _NW)
_seg_sum_b = _make_seg_sum(_EB // _NW, _EB // _NW)
_seg_sum_c = _make_seg_sum(_EC // _NW, _EC // _NW)


def _glif_body(pa_ref, pb_ref, pc_ref, v_ref, ext_ref, decay_ref,
               cf_ref, vth_ref, vreset_ref, el_ref, out_ref):
    rec = ((pa_ref[0:1, :_N] + pa_ref[0:1, _NPAD:_NPAD + _N])
           + (pb_ref[0:1, :_N] + pb_ref[0:1, _NPAD:_NPAD + _N])
           + (pc_ref[0:1, :_N] + pc_ref[0:1, _NPAD:_NPAD + _N]))
    new_v = decay_ref[...] * v_ref[...] + cf_ref[...] * (rec + ext_ref[...])
    v_scaled = (new_v - vth_ref[...]) / (vth_ref[...] - el_ref[...] + 1e-8)
    z = (v_scaled > 0.0).astype(jnp.float32)
    v_out = new_v * (1.0 - z) + vreset_ref[...] * z
    out_ref[0:1, :_N] = z
    out_ref[0:1, _N:] = v_out


def kernel(z_buf, v, ext_current, rec_weights, decay, current_factor,
           v_th, v_reset, e_l, rec_indices):
    zflat = jnp.pad(z_buf.reshape(-1), (0, _ZPAD - _ND))
    pa = _seg_sum_a(rec_indices[:_EA, 1], rec_indices[:_EA, 0],
                    rec_weights[:_EA], zflat)
    pb = _seg_sum_b(rec_indices[_EA:_EA + _EB, 1], rec_indices[_EA:_EA + _EB, 0],
                    rec_weights[_EA:_EA + _EB], zflat)
    pc = _seg_sum_c(rec_indices[_EA + _EB:, 1], rec_indices[_EA + _EB:, 0],
                    rec_weights[_EA + _EB:], zflat)
    return pl.pallas_call(
        _glif_body,
        out_shape=jax.ShapeDtypeStruct((1, 2 * _N), jnp.float32),
    )(pa[None, :], pb[None, :], pc[None, :], v, ext_current, decay[None, :],
      current_factor[None, :], v_th[None, :], v_reset[None, :], e_l[None, :])
